# Initial kernel scaffold; baseline (speedup 1.0000x reference)
import jax
import jax.numpy as jnp
from jax.experimental import pallas as pl


def kernel(x, edge_index, edge_weight, batch, class_labels, W1, b1, emb, W2, b2, W3, b3):
    raise NotImplementedError("write your pallas kernel here")


# speedup vs baseline: 1.4967x; 1.4967x over previous
import functools

import jax
import jax.numpy as jnp
from jax.experimental import pallas as pl
from jax.experimental.pallas import tpu as pltpu

N = 10000
E = 320000
D = 128
H = 256
C = 10
B = 64

_ROWS = 400
_NBLK = N // _ROWS


def _tail_body(agg_ref, batch_ref, cls_ref, W1_ref, b1_ref, emb_ref,
               W2a_ref, W2b_ref, b2_ref, W3_ref, b3_ref, out_ref,
               pooled_acc, cnt_acc):
    i = pl.program_id(0)

    @pl.when(i == 0)
    def _():
        pooled_acc[...] = jnp.zeros_like(pooled_acc)
        cnt_acc[...] = jnp.zeros_like(cnt_acc)

    conv = jnp.dot(agg_ref[...], W1_ref[...],
                   preferred_element_type=jnp.float32) + b1_ref[...]
    out = jnp.where(conv > 0, conv, 0.2 * conv)
    seg = batch_ref[...]
    onehot = (seg == jax.lax.broadcasted_iota(jnp.int32, (_ROWS, B), 1)
              ).astype(jnp.float32)
    pooled_acc[...] += jax.lax.dot_general(
        onehot, out, (((0,), (0,)), ((), ())),
        preferred_element_type=jnp.float32)
    cnt_acc[...] += jax.lax.dot_general(
        onehot, jnp.ones((_ROWS, 1), jnp.float32), (((0,), (0,)), ((), ())),
        preferred_element_type=jnp.float32)

    @pl.when(i == _NBLK - 1)
    def _():
        pooled = pooled_acc[...] / jnp.maximum(cnt_acc[...], 1.0)
        cls = cls_ref[...]
        oh_cls = (cls == jax.lax.broadcasted_iota(jnp.int32, (B, C), 1)
                  ).astype(jnp.float32)
        ce = jnp.dot(oh_cls, emb_ref[...],
                     preferred_element_type=jnp.float32)
        z = (jnp.dot(pooled, W2a_ref[...], preferred_element_type=jnp.float32)
             + jnp.dot(ce, W2b_ref[...], preferred_element_type=jnp.float32)
             + b2_ref[...])
        z = jnp.where(z > 0, z, 0.2 * z)
        res = jnp.dot(z, W3_ref[...],
                      preferred_element_type=jnp.float32) + b3_ref[...]
        out_ref[...] = res


def _dense_tail(agg, batch, class_labels, W1, b1, emb, W2, b2, W3, b3):
    batch2 = batch.astype(jnp.int32).reshape(N, 1)
    cls2 = class_labels.astype(jnp.int32).reshape(B, 1)
    W2a = W2[:H]
    W2b = W2[H:]
    W3p = jnp.pad(W3, ((0, 0), (0, 127)))
    b1r = b1.reshape(1, H)
    b2r = b2.reshape(1, H)
    b3r = b3.reshape(1, 1)
    out = pl.pallas_call(
        _tail_body,
        grid=(_NBLK,),
        in_specs=[
            pl.BlockSpec((_ROWS, D), lambda i: (i, 0)),
            pl.BlockSpec((_ROWS, 1), lambda i: (i, 0)),
            pl.BlockSpec((B, 1), lambda i: (0, 0)),
            pl.BlockSpec((D, H), lambda i: (0, 0)),
            pl.BlockSpec((1, H), lambda i: (0, 0)),
            pl.BlockSpec((C, H // 2), lambda i: (0, 0)),
            pl.BlockSpec((H, H), lambda i: (0, 0)),
            pl.BlockSpec((H // 2, H), lambda i: (0, 0)),
            pl.BlockSpec((1, H), lambda i: (0, 0)),
            pl.BlockSpec((H, 128), lambda i: (0, 0)),
            pl.BlockSpec((1, 1), lambda i: (0, 0)),
        ],
        out_specs=pl.BlockSpec((B, 128), lambda i: (0, 0)),
        out_shape=jax.ShapeDtypeStruct((B, 128), jnp.float32),
        scratch_shapes=[
            pltpu.VMEM((B, H), jnp.float32),
            pltpu.VMEM((B, 1), jnp.float32),
        ],
    )(agg, batch2, cls2, W1, b1r, emb, W2a, W2b, b2r, W3p, b3r)
    return out[:, :1]


def kernel(x, edge_index, edge_weight, batch, class_labels,
           W1, b1, emb, W2, b2, W3, b3):
    src = edge_index[0]
    dst = edge_index[1]
    ew = edge_weight
    deg = jnp.zeros((N,), jnp.float32).at[dst].add(ew) + 1.0
    dinv = jax.lax.rsqrt(deg)
    norm = dinv[src] * ew * dinv[dst]
    agg = (jnp.zeros((N, D), jnp.float32).at[dst].add(x[src] * norm[:, None])
           + x * (dinv * dinv)[:, None])
    return _dense_tail(agg, batch, class_labels, W1, b1, emb, W2, b2, W3, b3)


# speedup vs baseline: 12.7482x; 8.5175x over previous
import dataclasses
import functools

import jax
import jax.numpy as jnp
from jax.experimental import pallas as pl
from jax.experimental.pallas import tpu as pltpu
from jax.experimental.pallas import tpu_sc as plsc

N = 10000
E = 320000
D = 128
H = 256
C = 10
B = 64

PADN = 10240
NSC = 2
NT = 16
SLICE = PADN // NT
E2 = E // NSC
EPT_DEG = E // NT
EPT_AGG = E2 // NT
CH = 80

_ROWS = 400
_NBLK = N // _ROWS


def _sc_graph(x, src, dst, ew):
    mesh = plsc.VectorSubcoreMesh(core_axis_name="c", subcore_axis_name="s")
    cp = pltpu.CompilerParams()
    if "needs_layout_passes" in pltpu.CompilerParams.__dataclass_fields__:
        cp = dataclasses.replace(cp, needs_layout_passes=False)

    @functools.partial(
        pl.kernel,
        compiler_params=cp,
        out_type=[jax.ShapeDtypeStruct((NSC, PADN, D), jnp.float32),
                  jax.ShapeDtypeStruct((NSC, PADN), jnp.float32)],
        mesh=mesh,
        scratch_types=[
            pltpu.VMEM_SHARED((PADN, D), jnp.float32),
            pltpu.VMEM_SHARED((PADN,), jnp.float32),
            pltpu.VMEM_SHARED((PADN,), jnp.float32),
            pltpu.VMEM((CH, D), jnp.float32),
            pltpu.VMEM((PADN,), jnp.float32),
            pltpu.VMEM((CH,), jnp.int32),
            pltpu.VMEM((CH,), jnp.int32),
            pltpu.VMEM((CH,), jnp.float32),
            pltpu.VMEM((CH,), jnp.float32),
            pltpu.VMEM((SLICE,), jnp.float32),
        ],
    )
    def k(x_hbm, src_hbm, dst_hbm, ew_hbm, S_out, dinv_out,
          S_sh, deg_sh, dinv_sh, rows, dinv_t, srcb, dstb, ewb, normb, degb):
        c = jax.lax.axis_index("c")
        s = jax.lax.axis_index("s")
        nb = s * SLICE
        z16 = jnp.zeros((16,), jnp.float32)

        @pl.loop(0, SLICE, step=16)
        def _(j):
            degb[pl.ds(j, 16)] = z16

        pltpu.sync_copy(degb, deg_sh.at[pl.ds(nb, SLICE)])

        @pl.loop(0, CH)
        def _(r):
            for j in range(8):
                rows[r, pl.ds(j * 16, 16)] = z16

        for off in range(0, SLICE, CH):
            pltpu.sync_copy(rows, S_sh.at[pl.ds(nb + off, CH)])
        plsc.subcore_barrier()

        dbase = s * EPT_DEG

        @pl.loop(0, EPT_DEG // CH)
        def _(k_):
            eb = dbase + k_ * CH
            pltpu.sync_copy(dst_hbm.at[pl.ds(eb, CH)], dstb)
            pltpu.sync_copy(ew_hbm.at[pl.ds(eb, CH)], ewb)
            pltpu.sync_copy(ewb, deg_sh.at[dstb], add=True)

        plsc.subcore_barrier()

        pltpu.sync_copy(deg_sh.at[pl.ds(nb, SLICE)], degb)

        @pl.loop(0, SLICE, step=16)
        def _(j):
            dd = degb[pl.ds(j, 16)] + 1.0
            ii = plsc.bitcast(dd, jnp.int32)
            ii = 0x5F3759DF - (ii >> 1)
            y = plsc.bitcast(ii, jnp.float32)
            y = y * (1.5 - 0.5 * dd * y * y)
            y = y * (1.5 - 0.5 * dd * y * y)
            y = y * (1.5 - 0.5 * dd * y * y)
            degb[pl.ds(j, 16)] = y

        pltpu.sync_copy(degb, dinv_sh.at[pl.ds(nb, SLICE)])
        pltpu.sync_copy(degb, dinv_out.at[c, pl.ds(nb, SLICE)])
        plsc.subcore_barrier()
        pltpu.sync_copy(dinv_sh, dinv_t)

        abase = c * E2 + s * EPT_AGG

        @pl.loop(0, EPT_AGG // CH)
        def _(k_):
            eb = abase + k_ * CH
            pltpu.sync_copy(src_hbm.at[pl.ds(eb, CH)], srcb)
            pltpu.sync_copy(dst_hbm.at[pl.ds(eb, CH)], dstb)
            pltpu.sync_copy(ew_hbm.at[pl.ds(eb, CH)], ewb)
            pltpu.sync_copy(x_hbm.at[srcb], rows)

            @pl.loop(0, CH, step=16)
            def _(e0):
                si = srcb[pl.ds(e0, 16)]
                di = dstb[pl.ds(e0, 16)]
                ns = plsc.load_gather(dinv_t, [si])
                nd = plsc.load_gather(dinv_t, [di])
                normb[pl.ds(e0, 16)] = ewb[pl.ds(e0, 16)] * ns * nd

            @pl.loop(0, CH, step=16)
            def _(e0):
                v = normb[pl.ds(e0, 16)]
                for l in range(16):
                    sc_ = v[l]
                    e = e0 + l
                    for j in range(8):
                        rows[e, pl.ds(j * 16, 16)] = rows[e, pl.ds(j * 16, 16)] * sc_

            pltpu.sync_copy(rows, S_sh.at[dstb], add=True)

        plsc.subcore_barrier()

        pltpu.sync_copy(S_sh.at[pl.ds(nb, SLICE)], S_out.at[c, pl.ds(nb, SLICE)])

    return k(x, src, dst, ew)


def _tail_body(s0_ref, s1_ref, x_ref, dinv_ref, batch_ref, cls_ref,
               W1_ref, b1_ref, emb_ref, W2a_ref, W2b_ref, b2_ref,
               W3_ref, b3_ref, out_ref, pooled_acc, cnt_acc):
    i = pl.program_id(0)

    @pl.when(i == 0)
    def _():
        pooled_acc[...] = jnp.zeros_like(pooled_acc)
        cnt_acc[...] = jnp.zeros_like(cnt_acc)

    d = dinv_ref[...]
    agg = s0_ref[0] + s1_ref[0] + x_ref[...] * (d * d)
    conv = jnp.dot(agg, W1_ref[...],
                   preferred_element_type=jnp.float32) + b1_ref[...]
    out = jnp.where(conv > 0, conv, 0.2 * conv)
    seg = batch_ref[...]
    onehot = (seg == jax.lax.broadcasted_iota(jnp.int32, (_ROWS, B), 1)
              ).astype(jnp.float32)
    pooled_acc[...] += jax.lax.dot_general(
        onehot, out, (((0,), (0,)), ((), ())),
        preferred_element_type=jnp.float32)
    cnt_acc[...] += jax.lax.dot_general(
        onehot, jnp.ones((_ROWS, 1), jnp.float32), (((0,), (0,)), ((), ())),
        preferred_element_type=jnp.float32)

    @pl.when(i == _NBLK - 1)
    def _():
        pooled = pooled_acc[...] / jnp.maximum(cnt_acc[...], 1.0)
        cls = cls_ref[...]
        oh_cls = (cls == jax.lax.broadcasted_iota(jnp.int32, (B, C), 1)
                  ).astype(jnp.float32)
        ce = jnp.dot(oh_cls, emb_ref[...],
                     preferred_element_type=jnp.float32)
        z = (jnp.dot(pooled, W2a_ref[...], preferred_element_type=jnp.float32)
             + jnp.dot(ce, W2b_ref[...], preferred_element_type=jnp.float32)
             + b2_ref[...])
        z = jnp.where(z > 0, z, 0.2 * z)
        res = jnp.dot(z, W3_ref[...],
                      preferred_element_type=jnp.float32) + b3_ref[...]
        out_ref[...] = res


def _dense_tail(S, dinv_col, x, batch, class_labels, W1, b1, emb, W2, b2, W3, b3):
    batch2 = batch.astype(jnp.int32).reshape(N, 1)
    cls2 = class_labels.astype(jnp.int32).reshape(B, 1)
    W2a = W2[:H]
    W2b = W2[H:]
    W3p = jnp.pad(W3, ((0, 0), (0, 127)))
    b1r = b1.reshape(1, H)
    b2r = b2.reshape(1, H)
    b3r = b3.reshape(1, 1)
    out = pl.pallas_call(
        _tail_body,
        grid=(_NBLK,),
        in_specs=[
            pl.BlockSpec((1, _ROWS, D), lambda i: (0, i, 0)),
            pl.BlockSpec((1, _ROWS, D), lambda i: (1, i, 0)),
            pl.BlockSpec((_ROWS, D), lambda i: (i, 0)),
            pl.BlockSpec((_ROWS, 1), lambda i: (i, 0)),
            pl.BlockSpec((_ROWS, 1), lambda i: (i, 0)),
            pl.BlockSpec((B, 1), lambda i: (0, 0)),
            pl.BlockSpec((D, H), lambda i: (0, 0)),
            pl.BlockSpec((1, H), lambda i: (0, 0)),
            pl.BlockSpec((C, H // 2), lambda i: (0, 0)),
            pl.BlockSpec((H, H), lambda i: (0, 0)),
            pl.BlockSpec((H // 2, H), lambda i: (0, 0)),
            pl.BlockSpec((1, H), lambda i: (0, 0)),
            pl.BlockSpec((H, 128), lambda i: (0, 0)),
            pl.BlockSpec((1, 1), lambda i: (0, 0)),
        ],
        out_specs=pl.BlockSpec((B, 128), lambda i: (0, 0)),
        out_shape=jax.ShapeDtypeStruct((B, 128), jnp.float32),
        scratch_shapes=[
            pltpu.VMEM((B, H), jnp.float32),
            pltpu.VMEM((B, 1), jnp.float32),
        ],
    )(S, S, x, dinv_col, batch2, cls2, W1, b1r, emb, W2a, W2b, b2r, W3p, b3r)
    return out[:, :1]


def kernel(x, edge_index, edge_weight, batch, class_labels,
           W1, b1, emb, W2, b2, W3, b3):
    src = edge_index[0].astype(jnp.int32)
    dst = edge_index[1].astype(jnp.int32)
    ew = edge_weight
    S, dinv = _sc_graph(x, src, dst, ew)
    dinv_col = dinv[0, :N].reshape(N, 1)
    return _dense_tail(S, dinv_col, x, batch, class_labels,
                       W1, b1, emb, W2, b2, W3, b3)


# speedup vs baseline: 33.8674x; 2.6566x over previous
import dataclasses
import functools

import jax
import jax.numpy as jnp
from jax.experimental import pallas as pl
from jax.experimental.pallas import tpu as pltpu
from jax.experimental.pallas import tpu_sc as plsc

N = 10000
E = 320000
D = 128
H = 256
C = 10
B = 64

PADN = 10240
NSC = 2
NT = 16
SLICE = PADN // NT
E2 = E // NSC
EPT_DEG = E // NT
EPT_AGG = E2 // NT
CH = 80

_ROWS = 400
_NBLK = N // _ROWS


NCHE = E // CH
NCH_SC = NCHE // NSC
NCH_T = NCH_SC // NT
NCH_DT = NCHE // NT


def _sc_graph(x, pk):
    mesh = plsc.VectorSubcoreMesh(core_axis_name="c", subcore_axis_name="s")
    cp = pltpu.CompilerParams()
    if "needs_layout_passes" in pltpu.CompilerParams.__dataclass_fields__:
        cp = dataclasses.replace(cp, needs_layout_passes=False)

    @functools.partial(
        pl.kernel,
        compiler_params=cp,
        out_type=[jax.ShapeDtypeStruct((NSC, PADN, D), jnp.float32),
                  jax.ShapeDtypeStruct((NSC, PADN), jnp.float32)],
        mesh=mesh,
        scratch_types=[
            pltpu.VMEM_SHARED((PADN, D), jnp.float32),
            pltpu.VMEM_SHARED((PADN,), jnp.float32),
            pltpu.VMEM_SHARED((PADN,), jnp.float32),
            pltpu.VMEM((CH, D), jnp.float32),
            pltpu.VMEM((CH, D), jnp.float32),
            pltpu.VMEM((CH, D), jnp.float32),
            pltpu.VMEM((PADN,), jnp.float32),
            pltpu.VMEM((3, CH), jnp.int32),
            pltpu.VMEM((3, CH), jnp.int32),
            pltpu.VMEM((3, CH), jnp.int32),
            pltpu.VMEM((1, CH), jnp.int32),
            pltpu.VMEM((1, CH), jnp.int32),
            pltpu.VMEM((1, CH), jnp.int32),
            pltpu.VMEM((CH,), jnp.float32),
            pltpu.VMEM((CH,), jnp.float32),
            pltpu.VMEM((CH,), jnp.float32),
            pltpu.VMEM((SLICE,), jnp.float32),
            pltpu.SemaphoreType.DMA,
            pltpu.SemaphoreType.DMA,
            pltpu.SemaphoreType.DMA,
            pltpu.SemaphoreType.DMA,
            pltpu.SemaphoreType.DMA,
            pltpu.SemaphoreType.DMA,
            pltpu.SemaphoreType.DMA,
            pltpu.SemaphoreType.DMA,
            pltpu.SemaphoreType.DMA,
        ],
    )
    def k(x_hbm, pk_hbm, S_out, dinv_out,
          S_sh, deg_sh, dinv_sh, rows0, rows1, rows2, dinv_t,
          ib0, ib1, ib2, dg0, dg1, dg2, ef0, ef1, normb, degb,
          isem0, isem1, isem2, gsem0, gsem1, gsem2, ssem0, ssem1, ssem2):
        rowsL = (rows0, rows1, rows2)
        ibL = (ib0, ib1, ib2)
        dgL = (dg0, dg1, dg2)
        efL = (ef0, ef1)
        isemL = (isem0, isem1, isem2)
        gsemL = (gsem0, gsem1, gsem2)
        ssemL = (ssem0, ssem1, ssem2)
        c = jax.lax.axis_index("c")
        s = jax.lax.axis_index("s")
        nb = s * SLICE
        z16 = jnp.zeros((16,), jnp.float32)

        @pl.loop(0, SLICE, step=16)
        def _(j):
            degb[pl.ds(j, 16)] = z16

        pltpu.sync_copy(degb, deg_sh.at[pl.ds(nb, SLICE)])

        @pl.loop(0, CH)
        def _(r):
            for j in range(8):
                rows0[r, pl.ds(j * 16, 16)] = z16

        for off in range(0, SLICE, CH):
            pltpu.sync_copy(rows0, S_sh.at[pl.ds(nb + off, CH)])
        plsc.subcore_barrier()

        drow0 = s * NCH_DT

        def d_start_i(t, r):
            pltpu.async_copy(pk_hbm.at[drow0 + t], ibL[r], isemL[r])

        def d_wait_i(t, r):
            pltpu.make_async_copy(pk_hbm.at[drow0 + t], ibL[r], isemL[r]).wait()

        def d_start_s(r):
            pltpu.async_copy(efL[r], deg_sh.at[dgL[r].at[0]], ssemL[r],
                             add=True)

        def d_wait_s(r):
            pltpu.make_async_copy(efL[r], deg_sh.at[dgL[r].at[0]],
                                  ssemL[r]).wait()

        def d_body(t, r):
            d_wait_i(t, r)

            @pl.when(t >= 2)
            def _():
                d_wait_s(r)

            @pl.loop(0, CH, step=16)
            def _(j):
                dgL[r][0, pl.ds(j, 16)] = ibL[r][1, pl.ds(j, 16)]
                efL[r][pl.ds(j, 16)] = plsc.bitcast(
                    ibL[r][2, pl.ds(j, 16)], jnp.float32)

            @pl.when(t + 2 < NCH_DT)
            def _():
                d_start_i(t + 2, r)

            d_start_s(r)

        d_start_i(0, 0)
        d_start_i(1, 1)

        @pl.loop(0, NCH_DT, step=2)
        def _(t):
            d_body(t, 0)
            d_body(t + 1, 1)

        d_wait_s(0)
        d_wait_s(1)
        plsc.subcore_barrier()

        pltpu.sync_copy(deg_sh.at[pl.ds(nb, SLICE)], degb)

        @pl.loop(0, SLICE, step=16)
        def _(j):
            dd = degb[pl.ds(j, 16)] + 1.0
            ii = plsc.bitcast(dd, jnp.int32)
            ii = 0x5F3759DF - (ii >> 1)
            y = plsc.bitcast(ii, jnp.float32)
            y = y * (1.5 - 0.5 * dd * y * y)
            y = y * (1.5 - 0.5 * dd * y * y)
            y = y * (1.5 - 0.5 * dd * y * y)
            degb[pl.ds(j, 16)] = y

        pltpu.sync_copy(degb, dinv_sh.at[pl.ds(nb, SLICE)])
        pltpu.sync_copy(degb, dinv_out.at[c, pl.ds(nb, SLICE)])
        plsc.subcore_barrier()
        pltpu.sync_copy(dinv_sh, dinv_t)

        arow0 = c * NCH_SC + s * NCH_T

        def a_start_i(t, r):
            pltpu.async_copy(pk_hbm.at[arow0 + t], ibL[r], isemL[r])

        def a_wait_i(t, r):
            pltpu.make_async_copy(pk_hbm.at[arow0 + t], ibL[r], isemL[r]).wait()

        def a_start_g(r):
            pltpu.async_copy(x_hbm.at[ibL[r].at[0]], rowsL[r], gsemL[r])

        def a_wait_g(r):
            pltpu.make_async_copy(x_hbm.at[ibL[r].at[0]], rowsL[r],
                                  gsemL[r]).wait()

        def a_start_s(r):
            pltpu.async_copy(rowsL[r], S_sh.at[dgL[r].at[0]], ssemL[r],
                             add=True)

        def a_wait_s(r):
            pltpu.make_async_copy(rowsL[r], S_sh.at[dgL[r].at[0]],
                                  ssemL[r]).wait()

        def a_compute(r):
            @pl.loop(0, CH, step=16)
            def _(j):
                si = ibL[r][0, pl.ds(j, 16)]
                di = ibL[r][1, pl.ds(j, 16)]
                ewv = plsc.bitcast(ibL[r][2, pl.ds(j, 16)], jnp.float32)
                ns = plsc.load_gather(dinv_t, [si])
                nd = plsc.load_gather(dinv_t, [di])
                normb[pl.ds(j, 16)] = ewv * ns * nd
                dgL[r][0, pl.ds(j, 16)] = di

            @pl.loop(0, CH, step=16)
            def _(e0):
                v = normb[pl.ds(e0, 16)]
                for l in range(16):
                    sc_ = v[l]
                    e = e0 + l
                    for j in range(8):
                        rowsL[r][e, pl.ds(j * 16, 16)] = (
                            rowsL[r][e, pl.ds(j * 16, 16)] * sc_)

        def a_body(t, r):
            rn = (r + 1) % 3
            a_wait_i(t + 1, rn)

            @pl.when(t >= 2)
            def _():
                a_wait_s(rn)

            a_start_g(rn)
            a_wait_g(r)
            a_compute(r)
            a_start_s(r)

            @pl.when(t + 3 < NCH_T)
            def _():
                a_start_i(t + 3, r)

        a_start_i(0, 0)
        a_start_i(1, 1)
        a_wait_i(0, 0)
        a_start_g(0)
        a_start_i(2, 2)

        @pl.loop(0, NCH_T - 2, step=3)
        def _(t):
            a_body(t, 0)
            a_body(t + 1, 1)
            a_body(t + 2, 2)

        t0 = NCH_T - 2
        a_wait_i(t0 + 1, 1)
        a_wait_s(1)
        a_start_g(1)
        a_wait_g(0)
        a_compute(0)
        a_start_s(0)
        a_wait_g(1)
        a_compute(1)
        a_start_s(1)
        a_wait_s(2)
        a_wait_s(0)
        a_wait_s(1)
        plsc.subcore_barrier()

        pltpu.sync_copy(S_sh.at[pl.ds(nb, SLICE)], S_out.at[c, pl.ds(nb, SLICE)])

    return k(x, pk)


def _tail_body(s0_ref, s1_ref, x_ref, dinv_ref, batch_ref, cls_ref,
               W1_ref, b1_ref, emb_ref, W2a_ref, W2b_ref, b2_ref,
               W3_ref, b3_ref, out_ref, pooled_acc, cnt_acc):
    i = pl.program_id(0)

    @pl.when(i == 0)
    def _():
        pooled_acc[...] = jnp.zeros_like(pooled_acc)
        cnt_acc[...] = jnp.zeros_like(cnt_acc)

    d = dinv_ref[...]
    agg = s0_ref[0] + s1_ref[0] + x_ref[...] * (d * d)
    conv = jnp.dot(agg, W1_ref[...],
                   preferred_element_type=jnp.float32) + b1_ref[...]
    out = jnp.where(conv > 0, conv, 0.2 * conv)
    seg = batch_ref[...]
    onehot = (seg == jax.lax.broadcasted_iota(jnp.int32, (_ROWS, B), 1)
              ).astype(jnp.float32)
    pooled_acc[...] += jax.lax.dot_general(
        onehot, out, (((0,), (0,)), ((), ())),
        preferred_element_type=jnp.float32)
    cnt_acc[...] += jax.lax.dot_general(
        onehot, jnp.ones((_ROWS, 1), jnp.float32), (((0,), (0,)), ((), ())),
        preferred_element_type=jnp.float32)

    @pl.when(i == _NBLK - 1)
    def _():
        pooled = pooled_acc[...] / jnp.maximum(cnt_acc[...], 1.0)
        cls = cls_ref[...]
        oh_cls = (cls == jax.lax.broadcasted_iota(jnp.int32, (B, C), 1)
                  ).astype(jnp.float32)
        ce = jnp.dot(oh_cls, emb_ref[...],
                     preferred_element_type=jnp.float32)
        z = (jnp.dot(pooled, W2a_ref[...], preferred_element_type=jnp.float32)
             + jnp.dot(ce, W2b_ref[...], preferred_element_type=jnp.float32)
             + b2_ref[...])
        z = jnp.where(z > 0, z, 0.2 * z)
        res = jnp.dot(z, W3_ref[...],
                      preferred_element_type=jnp.float32) + b3_ref[...]
        out_ref[...] = res


def _dense_tail(S, dinv_col, x, batch, class_labels, W1, b1, emb, W2, b2, W3, b3):
    batch2 = batch.astype(jnp.int32).reshape(N, 1)
    cls2 = class_labels.astype(jnp.int32).reshape(B, 1)
    W2a = W2[:H]
    W2b = W2[H:]
    W3p = jnp.pad(W3, ((0, 0), (0, 127)))
    b1r = b1.reshape(1, H)
    b2r = b2.reshape(1, H)
    b3r = b3.reshape(1, 1)
    out = pl.pallas_call(
        _tail_body,
        grid=(_NBLK,),
        in_specs=[
            pl.BlockSpec((1, _ROWS, D), lambda i: (0, i, 0)),
            pl.BlockSpec((1, _ROWS, D), lambda i: (1, i, 0)),
            pl.BlockSpec((_ROWS, D), lambda i: (i, 0)),
            pl.BlockSpec((_ROWS, 1), lambda i: (i, 0)),
            pl.BlockSpec((_ROWS, 1), lambda i: (i, 0)),
            pl.BlockSpec((B, 1), lambda i: (0, 0)),
            pl.BlockSpec((D, H), lambda i: (0, 0)),
            pl.BlockSpec((1, H), lambda i: (0, 0)),
            pl.BlockSpec((C, H // 2), lambda i: (0, 0)),
            pl.BlockSpec((H, H), lambda i: (0, 0)),
            pl.BlockSpec((H // 2, H), lambda i: (0, 0)),
            pl.BlockSpec((1, H), lambda i: (0, 0)),
            pl.BlockSpec((H, 128), lambda i: (0, 0)),
            pl.BlockSpec((1, 1), lambda i: (0, 0)),
        ],
        out_specs=pl.BlockSpec((B, 128), lambda i: (0, 0)),
        out_shape=jax.ShapeDtypeStruct((B, 128), jnp.float32),
        scratch_shapes=[
            pltpu.VMEM((B, H), jnp.float32),
            pltpu.VMEM((B, 1), jnp.float32),
        ],
    )(S, S, x, dinv_col, batch2, cls2, W1, b1r, emb, W2a, W2b, b2r, W3p, b3r)
    return out[:, :1]


def kernel(x, edge_index, edge_weight, batch, class_labels,
           W1, b1, emb, W2, b2, W3, b3):
    src = edge_index[0].astype(jnp.int32)
    dst = edge_index[1].astype(jnp.int32)
    ewi = jax.lax.bitcast_convert_type(edge_weight, jnp.int32)
    pk = jnp.stack([src.reshape(NCHE, CH), dst.reshape(NCHE, CH),
                    ewi.reshape(NCHE, CH)], axis=1)
    S, dinv = _sc_graph(x, pk)
    dinv_col = dinv[0, :N].reshape(N, 1)
    return _dense_tail(S, dinv_col, x, batch, class_labels,
                       W1, b1, emb, W2, b2, W3, b3)


# speedup vs baseline: 38.0107x; 1.1223x over previous
import dataclasses
import functools

import jax
import jax.numpy as jnp
from jax.experimental import pallas as pl
from jax.experimental.pallas import tpu as pltpu
from jax.experimental.pallas import tpu_sc as plsc

N = 10000
E = 320000
D = 128
H = 256
C = 10
B = 64

PADN = 10240
NSC = 2
NT = 16
SLICE = PADN // NT
E2 = E // NSC
EPT_DEG = E // NT
EPT_AGG = E2 // NT
CH = 80

_ROWS = 1000
_NBLK = N // _ROWS


NCHE = E // CH
NCH_SC = NCHE // NSC
NCH_T = NCH_SC // NT
NCH_DT = NCHE // NT


def _sc_graph(x, esrc, edst, ew):
    mesh = plsc.VectorSubcoreMesh(core_axis_name="c", subcore_axis_name="s")
    cp = pltpu.CompilerParams()
    if "needs_layout_passes" in pltpu.CompilerParams.__dataclass_fields__:
        cp = dataclasses.replace(cp, needs_layout_passes=False)

    @functools.partial(
        pl.kernel,
        compiler_params=cp,
        out_type=[jax.ShapeDtypeStruct((NSC, PADN, D), jnp.float32),
                  jax.ShapeDtypeStruct((NSC, PADN), jnp.float32)],
        mesh=mesh,
        scratch_types=[
            pltpu.VMEM_SHARED((PADN, D), jnp.float32),
            pltpu.VMEM_SHARED((PADN,), jnp.float32),
            pltpu.VMEM_SHARED((PADN,), jnp.float32),
            pltpu.VMEM((CH, D), jnp.float32),
            pltpu.VMEM((CH, D), jnp.float32),
            pltpu.VMEM((CH, D), jnp.float32),
            pltpu.VMEM((PADN,), jnp.float32),
            pltpu.VMEM((2, CH), jnp.int32),
            pltpu.VMEM((2, CH), jnp.int32),
            pltpu.VMEM((2, CH), jnp.int32),
            pltpu.VMEM((CH,), jnp.float32),
            pltpu.VMEM((CH,), jnp.float32),
            pltpu.VMEM((CH,), jnp.float32),
            pltpu.VMEM((1, CH), jnp.int32),
            pltpu.VMEM((1, CH), jnp.int32),
            pltpu.VMEM((1, CH), jnp.int32),
            pltpu.VMEM((CH,), jnp.float32),
            pltpu.VMEM((CH,), jnp.float32),
            pltpu.VMEM((CH,), jnp.float32),
            pltpu.VMEM((SLICE,), jnp.float32),
            pltpu.SemaphoreType.DMA,
            pltpu.SemaphoreType.DMA,
            pltpu.SemaphoreType.DMA,
            pltpu.SemaphoreType.DMA,
            pltpu.SemaphoreType.DMA,
            pltpu.SemaphoreType.DMA,
            pltpu.SemaphoreType.DMA,
            pltpu.SemaphoreType.DMA,
            pltpu.SemaphoreType.DMA,
        ],
    )
    def k(x_hbm, src_hbm, dst_hbm, ew_hbm, S_out, dinv_out,
          S_sh, deg_sh, dinv_sh, rows0, rows1, rows2, dinv_t,
          ib0, ib1, ib2, ewb0, ewb1, ewb2, dg0, dg1, dg2, ef0, ef1,
          normb, degb,
          isem0, isem1, isem2, gsem0, gsem1, gsem2, ssem0, ssem1, ssem2):
        rowsL = (rows0, rows1, rows2)
        ibL = (ib0, ib1, ib2)
        ewbL = (ewb0, ewb1, ewb2)
        dgL = (dg0, dg1, dg2)
        efL = (ef0, ef1)
        isemL = (isem0, isem1, isem2)
        gsemL = (gsem0, gsem1, gsem2)
        ssemL = (ssem0, ssem1, ssem2)
        c = jax.lax.axis_index("c")
        s = jax.lax.axis_index("s")
        nb = s * SLICE
        z16 = jnp.zeros((16,), jnp.float32)

        @pl.loop(0, SLICE, step=16)
        def _(j):
            degb[pl.ds(j, 16)] = z16

        pltpu.sync_copy(degb, deg_sh.at[pl.ds(nb, SLICE)])

        @pl.loop(0, CH)
        def _(r):
            for j in range(8):
                rows0[r, pl.ds(j * 16, 16)] = z16

        for off in range(0, SLICE, CH):
            pltpu.sync_copy(rows0, S_sh.at[pl.ds(nb + off, CH)])
        plsc.subcore_barrier()

        drow0 = s * NCH_DT

        def d_start_i(t, r):
            e0 = (drow0 + t) * CH
            pltpu.async_copy(src_hbm.at[pl.ds(e0, CH)], ibL[r].at[0], isemL[r])
            pltpu.async_copy(dst_hbm.at[pl.ds(e0, CH)], ibL[r].at[1], isemL[r])
            pltpu.async_copy(ew_hbm.at[pl.ds(e0, CH)], ewbL[r], isemL[r])

        def d_wait_i(t, r):
            e0 = (drow0 + t) * CH
            pltpu.make_async_copy(src_hbm.at[pl.ds(e0, CH)], ibL[r].at[0],
                                  isemL[r]).wait()
            pltpu.make_async_copy(dst_hbm.at[pl.ds(e0, CH)], ibL[r].at[1],
                                  isemL[r]).wait()
            pltpu.make_async_copy(ew_hbm.at[pl.ds(e0, CH)], ewbL[r],
                                  isemL[r]).wait()

        def d_start_s(r):
            pltpu.async_copy(efL[r], deg_sh.at[dgL[r].at[0]], ssemL[r],
                             add=True)

        def d_wait_s(r):
            pltpu.make_async_copy(efL[r], deg_sh.at[dgL[r].at[0]],
                                  ssemL[r]).wait()

        def d_body(t, r):
            d_wait_i(t, r)

            @pl.when(t >= 2)
            def _():
                d_wait_s(r)

            @pl.loop(0, CH, step=16)
            def _(j):
                dgL[r][0, pl.ds(j, 16)] = ibL[r][1, pl.ds(j, 16)]
                efL[r][pl.ds(j, 16)] = ewbL[r][pl.ds(j, 16)]

            @pl.when(t + 2 < NCH_DT)
            def _():
                d_start_i(t + 2, r)

            d_start_s(r)

        d_start_i(0, 0)
        d_start_i(1, 1)

        @pl.loop(0, NCH_DT, step=2)
        def _(t):
            d_body(t, 0)
            d_body(t + 1, 1)

        d_wait_s(0)
        d_wait_s(1)
        plsc.subcore_barrier()

        pltpu.sync_copy(deg_sh.at[pl.ds(nb, SLICE)], degb)

        @pl.loop(0, SLICE, step=16)
        def _(j):
            dd = degb[pl.ds(j, 16)] + 1.0
            ii = plsc.bitcast(dd, jnp.int32)
            ii = 0x5F3759DF - (ii >> 1)
            y = plsc.bitcast(ii, jnp.float32)
            y = y * (1.5 - 0.5 * dd * y * y)
            y = y * (1.5 - 0.5 * dd * y * y)
            y = y * (1.5 - 0.5 * dd * y * y)
            degb[pl.ds(j, 16)] = y

        pltpu.sync_copy(degb, dinv_sh.at[pl.ds(nb, SLICE)])
        pltpu.sync_copy(degb, dinv_out.at[c, pl.ds(nb, SLICE)])
        plsc.subcore_barrier()
        pltpu.sync_copy(dinv_sh, dinv_t)

        arow0 = c * NCH_SC + s * NCH_T

        def a_start_i(t, r):
            e0 = (arow0 + t) * CH
            pltpu.async_copy(src_hbm.at[pl.ds(e0, CH)], ibL[r].at[0], isemL[r])
            pltpu.async_copy(dst_hbm.at[pl.ds(e0, CH)], ibL[r].at[1], isemL[r])
            pltpu.async_copy(ew_hbm.at[pl.ds(e0, CH)], ewbL[r], isemL[r])

        def a_wait_i(t, r):
            e0 = (arow0 + t) * CH
            pltpu.make_async_copy(src_hbm.at[pl.ds(e0, CH)], ibL[r].at[0],
                                  isemL[r]).wait()
            pltpu.make_async_copy(dst_hbm.at[pl.ds(e0, CH)], ibL[r].at[1],
                                  isemL[r]).wait()
            pltpu.make_async_copy(ew_hbm.at[pl.ds(e0, CH)], ewbL[r],
                                  isemL[r]).wait()

        def a_start_g(r):
            pltpu.async_copy(x_hbm.at[ibL[r].at[0]], rowsL[r], gsemL[r])

        def a_wait_g(r):
            pltpu.make_async_copy(x_hbm.at[ibL[r].at[0]], rowsL[r],
                                  gsemL[r]).wait()

        def a_start_s(r):
            pltpu.async_copy(rowsL[r], S_sh.at[dgL[r].at[0]], ssemL[r],
                             add=True)

        def a_wait_s(r):
            pltpu.make_async_copy(rowsL[r], S_sh.at[dgL[r].at[0]],
                                  ssemL[r]).wait()

        def a_compute(r):
            @pl.loop(0, CH, step=16)
            def _(j):
                si = ibL[r][0, pl.ds(j, 16)]
                di = ibL[r][1, pl.ds(j, 16)]
                ewv = ewbL[r][pl.ds(j, 16)]
                ns = plsc.load_gather(dinv_t, [si])
                nd = plsc.load_gather(dinv_t, [di])
                normb[pl.ds(j, 16)] = ewv * ns * nd
                dgL[r][0, pl.ds(j, 16)] = di

            @pl.loop(0, CH, step=16)
            def _(e0):
                v = normb[pl.ds(e0, 16)]
                for l in range(16):
                    sc_ = v[l]
                    e = e0 + l
                    for j in range(8):
                        rowsL[r][e, pl.ds(j * 16, 16)] = (
                            rowsL[r][e, pl.ds(j * 16, 16)] * sc_)

        def a_body(t, r):
            rn = (r + 1) % 3
            a_wait_i(t + 1, rn)

            @pl.when(t >= 2)
            def _():
                a_wait_s(rn)

            a_start_g(rn)
            a_wait_g(r)
            a_compute(r)
            a_start_s(r)

            @pl.when(t + 3 < NCH_T)
            def _():
                a_start_i(t + 3, r)

        a_start_i(0, 0)
        a_start_i(1, 1)
        a_wait_i(0, 0)
        a_start_g(0)
        a_start_i(2, 2)

        @pl.loop(0, NCH_T - 2, step=3)
        def _(t):
            a_body(t, 0)
            a_body(t + 1, 1)
            a_body(t + 2, 2)

        t0 = NCH_T - 2
        a_wait_i(t0 + 1, 1)
        a_wait_s(1)
        a_start_g(1)
        a_wait_g(0)
        a_compute(0)
        a_start_s(0)
        a_wait_g(1)
        a_compute(1)
        a_start_s(1)
        a_wait_s(2)
        a_wait_s(0)
        a_wait_s(1)
        plsc.subcore_barrier()

        pltpu.sync_copy(S_sh.at[pl.ds(nb, SLICE)], S_out.at[c, pl.ds(nb, SLICE)])

    return k(x, esrc, edst, ew)


def _tail_body(s0_ref, s1_ref, x_ref, dinv_ref, batch_ref, cls_ref,
               W1_ref, b1_ref, emb_ref, W2a_ref, W2b_ref, b2_ref,
               W3_ref, b3_ref, out_ref, pooled_acc, cnt_acc):
    i = pl.program_id(0)

    @pl.when(i == 0)
    def _():
        pooled_acc[...] = jnp.zeros_like(pooled_acc)
        cnt_acc[...] = jnp.zeros_like(cnt_acc)

    d = dinv_ref[...]
    agg = s0_ref[0] + s1_ref[0] + x_ref[...] * (d * d)
    conv = jnp.dot(agg, W1_ref[...],
                   preferred_element_type=jnp.float32) + b1_ref[...]
    out = jnp.where(conv > 0, conv, 0.2 * conv)
    seg = batch_ref[...]
    onehot = (seg == jax.lax.broadcasted_iota(jnp.int32, (_ROWS, B), 1)
              ).astype(jnp.float32)
    pooled_acc[...] += jax.lax.dot_general(
        onehot, out, (((0,), (0,)), ((), ())),
        preferred_element_type=jnp.float32)
    cnt_acc[...] += jax.lax.dot_general(
        onehot, jnp.ones((_ROWS, 1), jnp.float32), (((0,), (0,)), ((), ())),
        preferred_element_type=jnp.float32)

    @pl.when(i == _NBLK - 1)
    def _():
        pooled = pooled_acc[...] / jnp.maximum(cnt_acc[...], 1.0)
        cls = cls_ref[...]
        oh_cls = (cls == jax.lax.broadcasted_iota(jnp.int32, (B, C), 1)
                  ).astype(jnp.float32)
        ce = jnp.dot(oh_cls, emb_ref[...],
                     preferred_element_type=jnp.float32)
        z = (jnp.dot(pooled, W2a_ref[...], preferred_element_type=jnp.float32)
             + jnp.dot(ce, W2b_ref[...], preferred_element_type=jnp.float32)
             + b2_ref[...])
        z = jnp.where(z > 0, z, 0.2 * z)
        res = jnp.dot(z, W3_ref[...],
                      preferred_element_type=jnp.float32) + b3_ref[...]
        out_ref[...] = res


def _dense_tail(S, dinv_col, x, batch, class_labels, W1, b1, emb, W2, b2, W3, b3):
    batch2 = batch.astype(jnp.int32).reshape(N, 1)
    cls2 = class_labels.astype(jnp.int32).reshape(B, 1)
    W2a = W2[:H]
    W2b = W2[H:]
    W3p = jnp.pad(W3, ((0, 0), (0, 127)))
    b1r = b1.reshape(1, H)
    b2r = b2.reshape(1, H)
    b3r = b3.reshape(1, 1)
    out = pl.pallas_call(
        _tail_body,
        grid=(_NBLK,),
        in_specs=[
            pl.BlockSpec((1, _ROWS, D), lambda i: (0, i, 0)),
            pl.BlockSpec((1, _ROWS, D), lambda i: (1, i, 0)),
            pl.BlockSpec((_ROWS, D), lambda i: (i, 0)),
            pl.BlockSpec((_ROWS, 1), lambda i: (i, 0)),
            pl.BlockSpec((_ROWS, 1), lambda i: (i, 0)),
            pl.BlockSpec((B, 1), lambda i: (0, 0)),
            pl.BlockSpec((D, H), lambda i: (0, 0)),
            pl.BlockSpec((1, H), lambda i: (0, 0)),
            pl.BlockSpec((C, H // 2), lambda i: (0, 0)),
            pl.BlockSpec((H, H), lambda i: (0, 0)),
            pl.BlockSpec((H // 2, H), lambda i: (0, 0)),
            pl.BlockSpec((1, H), lambda i: (0, 0)),
            pl.BlockSpec((H, 128), lambda i: (0, 0)),
            pl.BlockSpec((1, 1), lambda i: (0, 0)),
        ],
        out_specs=pl.BlockSpec((B, 128), lambda i: (0, 0)),
        out_shape=jax.ShapeDtypeStruct((B, 128), jnp.float32),
        scratch_shapes=[
            pltpu.VMEM((B, H), jnp.float32),
            pltpu.VMEM((B, 1), jnp.float32),
        ],
    )(S, S, x, dinv_col, batch2, cls2, W1, b1r, emb, W2a, W2b, b2r, W3p, b3r)
    return out[:, :1]


def kernel(x, edge_index, edge_weight, batch, class_labels,
           W1, b1, emb, W2, b2, W3, b3):
    esrc = edge_index[0].astype(jnp.int32)
    edst = edge_index[1].astype(jnp.int32)
    S, dinv = _sc_graph(x, esrc, edst, edge_weight)
    dinv_col = dinv[0, :N].reshape(N, 1)
    return _dense_tail(S, dinv_col, x, batch, class_labels,
                       W1, b1, emb, W2, b2, W3, b3)


# speedup vs baseline: 38.3885x; 1.0099x over previous
import dataclasses
import functools

import jax
import jax.numpy as jnp
from jax.experimental import pallas as pl
from jax.experimental.pallas import tpu as pltpu
from jax.experimental.pallas import tpu_sc as plsc

N = 10000
E = 320000
D = 128
H = 256
C = 10
B = 64

PADN = 10240
NSC = 2
NT = 16
SLICE = PADN // NT
E2 = E // NSC
EPT_DEG = E // NT
EPT_AGG = E2 // NT
CH = 80

_ROWS = 1000
_NBLK = N // _ROWS


NCHE = E // CH
NCH_SC = NCHE // NSC
NCH_T = NCH_SC // NT
NCH_DT = NCHE // NT


def _sc_graph(x, esrc, edst, ew):
    mesh = plsc.VectorSubcoreMesh(core_axis_name="c", subcore_axis_name="s")
    cp = pltpu.CompilerParams()
    if "needs_layout_passes" in pltpu.CompilerParams.__dataclass_fields__:
        cp = dataclasses.replace(cp, needs_layout_passes=False)

    @functools.partial(
        pl.kernel,
        compiler_params=cp,
        out_type=[jax.ShapeDtypeStruct((NSC, PADN, D), jnp.float32),
                  jax.ShapeDtypeStruct((NSC, PADN), jnp.float32)],
        mesh=mesh,
        scratch_types=[
            pltpu.VMEM_SHARED((PADN, D), jnp.float32),
            pltpu.VMEM_SHARED((PADN,), jnp.float32),
            pltpu.VMEM_SHARED((PADN,), jnp.float32),
            pltpu.VMEM((CH, D), jnp.float32),
            pltpu.VMEM((CH, D), jnp.float32),
            pltpu.VMEM((CH, D), jnp.float32),
            pltpu.VMEM((PADN,), jnp.float32),
            pltpu.VMEM((CH,), jnp.int32),
            pltpu.VMEM((CH,), jnp.int32),
            pltpu.VMEM((CH,), jnp.int32),
            pltpu.VMEM((CH,), jnp.int32),
            pltpu.VMEM((CH,), jnp.int32),
            pltpu.VMEM((CH,), jnp.int32),
            pltpu.VMEM((CH,), jnp.float32),
            pltpu.VMEM((CH,), jnp.float32),
            pltpu.VMEM((CH,), jnp.float32),
            pltpu.VMEM((1, CH), jnp.int32),
            pltpu.VMEM((1, CH), jnp.int32),
            pltpu.VMEM((1, CH), jnp.int32),
            pltpu.VMEM((CH,), jnp.float32),
            pltpu.VMEM((CH,), jnp.float32),
            pltpu.VMEM((CH,), jnp.float32),
            pltpu.VMEM((SLICE,), jnp.float32),
            pltpu.SemaphoreType.DMA,
            pltpu.SemaphoreType.DMA,
            pltpu.SemaphoreType.DMA,
            pltpu.SemaphoreType.DMA,
            pltpu.SemaphoreType.DMA,
            pltpu.SemaphoreType.DMA,
            pltpu.SemaphoreType.DMA,
            pltpu.SemaphoreType.DMA,
            pltpu.SemaphoreType.DMA,
        ],
    )
    def k(x_hbm, src_hbm, dst_hbm, ew_hbm, S_out, dinv_out,
          S_sh, deg_sh, dinv_sh, rows0, rows1, rows2, dinv_t,
          sb0, sb1, sb2, db0, db1, db2, ewb0, ewb1, ewb2,
          dg0, dg1, dg2, ef0, ef1, normb, degb,
          isem0, isem1, isem2, gsem0, gsem1, gsem2, ssem0, ssem1, ssem2):
        rowsL = (rows0, rows1, rows2)
        sbL = (sb0, sb1, sb2)
        dbL = (db0, db1, db2)
        ewbL = (ewb0, ewb1, ewb2)
        dgL = (dg0, dg1, dg2)
        efL = (ef0, ef1)
        isemL = (isem0, isem1, isem2)
        gsemL = (gsem0, gsem1, gsem2)
        ssemL = (ssem0, ssem1, ssem2)
        c = jax.lax.axis_index("c")
        s = jax.lax.axis_index("s")
        nb = s * SLICE
        z16 = jnp.zeros((16,), jnp.float32)

        @pl.loop(0, SLICE, step=16)
        def _(j):
            degb[pl.ds(j, 16)] = z16

        pltpu.sync_copy(degb, deg_sh.at[pl.ds(nb, SLICE)])

        @pl.loop(0, CH)
        def _(r):
            for j in range(8):
                rows0[r, pl.ds(j * 16, 16)] = z16

        for off in range(0, SLICE, CH):
            pltpu.sync_copy(rows0, S_sh.at[pl.ds(nb + off, CH)])
        plsc.subcore_barrier()

        drow0 = s * NCH_DT

        def d_start_i(t, r):
            e0 = (drow0 + t) * CH
            pltpu.async_copy(dst_hbm.at[pl.ds(e0, CH)], dbL[r], isemL[r])
            pltpu.async_copy(ew_hbm.at[pl.ds(e0, CH)], ewbL[r], isemL[r])

        def d_wait_i(t, r):
            e0 = (drow0 + t) * CH
            pltpu.make_async_copy(dst_hbm.at[pl.ds(e0, CH)], dbL[r],
                                  isemL[r]).wait()
            pltpu.make_async_copy(ew_hbm.at[pl.ds(e0, CH)], ewbL[r],
                                  isemL[r]).wait()

        def d_start_s(r):
            pltpu.async_copy(efL[r], deg_sh.at[dgL[r].at[0]], ssemL[r],
                             add=True)

        def d_wait_s(r):
            pltpu.make_async_copy(efL[r], deg_sh.at[dgL[r].at[0]],
                                  ssemL[r]).wait()

        def d_body(t, r):
            d_wait_i(t, r)

            @pl.when(t >= 2)
            def _():
                d_wait_s(r)

            @pl.loop(0, CH, step=16)
            def _(j):
                dgL[r][0, pl.ds(j, 16)] = dbL[r][pl.ds(j, 16)]
                efL[r][pl.ds(j, 16)] = ewbL[r][pl.ds(j, 16)]

            @pl.when(t + 2 < NCH_DT)
            def _():
                d_start_i(t + 2, r)

            d_start_s(r)

        d_start_i(0, 0)
        d_start_i(1, 1)

        @pl.loop(0, NCH_DT, step=2)
        def _(t):
            d_body(t, 0)
            d_body(t + 1, 1)

        d_wait_s(0)
        d_wait_s(1)
        plsc.subcore_barrier()

        pltpu.sync_copy(deg_sh.at[pl.ds(nb, SLICE)], degb)

        @pl.loop(0, SLICE, step=16)
        def _(j):
            dd = degb[pl.ds(j, 16)] + 1.0
            ii = plsc.bitcast(dd, jnp.int32)
            ii = 0x5F3759DF - (ii >> 1)
            y = plsc.bitcast(ii, jnp.float32)
            y = y * (1.5 - 0.5 * dd * y * y)
            y = y * (1.5 - 0.5 * dd * y * y)
            y = y * (1.5 - 0.5 * dd * y * y)
            degb[pl.ds(j, 16)] = y

        pltpu.sync_copy(degb, dinv_sh.at[pl.ds(nb, SLICE)])
        pltpu.sync_copy(degb, dinv_out.at[c, pl.ds(nb, SLICE)])
        plsc.subcore_barrier()
        pltpu.sync_copy(dinv_sh, dinv_t)

        arow0 = c * NCH_SC + s * NCH_T

        def a_start_i(t, r):
            e0 = (arow0 + t) * CH
            pltpu.async_copy(src_hbm.at[pl.ds(e0, CH)], sbL[r], isemL[r])
            pltpu.async_copy(dst_hbm.at[pl.ds(e0, CH)], dbL[r], isemL[r])
            pltpu.async_copy(ew_hbm.at[pl.ds(e0, CH)], ewbL[r], isemL[r])

        def a_wait_i(t, r):
            e0 = (arow0 + t) * CH
            pltpu.make_async_copy(src_hbm.at[pl.ds(e0, CH)], sbL[r],
                                  isemL[r]).wait()
            pltpu.make_async_copy(dst_hbm.at[pl.ds(e0, CH)], dbL[r],
                                  isemL[r]).wait()
            pltpu.make_async_copy(ew_hbm.at[pl.ds(e0, CH)], ewbL[r],
                                  isemL[r]).wait()

        def a_start_g(r):
            pltpu.async_copy(x_hbm.at[sbL[r]], rowsL[r], gsemL[r])

        def a_wait_g(r):
            pltpu.make_async_copy(x_hbm.at[sbL[r]], rowsL[r],
                                  gsemL[r]).wait()

        def a_start_s(r):
            pltpu.async_copy(rowsL[r], S_sh.at[dgL[r].at[0]], ssemL[r],
                             add=True)

        def a_wait_s(r):
            pltpu.make_async_copy(rowsL[r], S_sh.at[dgL[r].at[0]],
                                  ssemL[r]).wait()

        def a_compute(r):
            @pl.loop(0, CH, step=16)
            def _(j):
                si = sbL[r][pl.ds(j, 16)]
                di = dbL[r][pl.ds(j, 16)]
                ewv = ewbL[r][pl.ds(j, 16)]
                ns = plsc.load_gather(dinv_t, [si])
                nd = plsc.load_gather(dinv_t, [di])
                normb[pl.ds(j, 16)] = ewv * ns * nd
                dgL[r][0, pl.ds(j, 16)] = di

            @pl.loop(0, CH, step=16)
            def _(e0):
                v = normb[pl.ds(e0, 16)]
                for l in range(16):
                    sc_ = v[l]
                    e = e0 + l
                    for j in range(8):
                        rowsL[r][e, pl.ds(j * 16, 16)] = (
                            rowsL[r][e, pl.ds(j * 16, 16)] * sc_)

        def a_body(t, r):
            rn = (r + 1) % 3
            a_wait_i(t + 1, rn)

            @pl.when(t >= 2)
            def _():
                a_wait_s(rn)

            a_start_g(rn)
            a_wait_g(r)
            a_compute(r)
            a_start_s(r)

            @pl.when(t + 3 < NCH_T)
            def _():
                a_start_i(t + 3, r)

        a_start_i(0, 0)
        a_start_i(1, 1)
        a_wait_i(0, 0)
        a_start_g(0)
        a_start_i(2, 2)

        @pl.loop(0, NCH_T - 2, step=3)
        def _(t):
            a_body(t, 0)
            a_body(t + 1, 1)
            a_body(t + 2, 2)

        t0 = NCH_T - 2
        a_wait_i(t0 + 1, 1)
        a_wait_s(1)
        a_start_g(1)
        a_wait_g(0)
        a_compute(0)
        a_start_s(0)
        a_wait_g(1)
        a_compute(1)
        a_start_s(1)
        a_wait_s(2)
        a_wait_s(0)
        a_wait_s(1)
        plsc.subcore_barrier()

        pltpu.sync_copy(S_sh.at[pl.ds(nb, SLICE)], S_out.at[c, pl.ds(nb, SLICE)])

    return k(x, esrc, edst, ew)


def _tail_body(s0_ref, s1_ref, x_ref, dinv_ref, batch_ref, cls_ref,
               W1_ref, b1_ref, emb_ref, W2a_ref, W2b_ref, b2_ref,
               W3_ref, b3_ref, out_ref, pooled_acc, cnt_acc):
    i = pl.program_id(0)

    @pl.when(i == 0)
    def _():
        pooled_acc[...] = jnp.zeros_like(pooled_acc)
        cnt_acc[...] = jnp.zeros_like(cnt_acc)

    d = dinv_ref[...]
    agg = s0_ref[0] + s1_ref[0] + x_ref[...] * (d * d)
    conv = jnp.dot(agg, W1_ref[...],
                   preferred_element_type=jnp.float32) + b1_ref[...]
    out = jnp.where(conv > 0, conv, 0.2 * conv)
    seg = batch_ref[...]
    onehot = (seg == jax.lax.broadcasted_iota(jnp.int32, (_ROWS, B), 1)
              ).astype(jnp.float32)
    pooled_acc[...] += jax.lax.dot_general(
        onehot, out, (((0,), (0,)), ((), ())),
        preferred_element_type=jnp.float32)
    cnt_acc[...] += jax.lax.dot_general(
        onehot, jnp.ones((_ROWS, 1), jnp.float32), (((0,), (0,)), ((), ())),
        preferred_element_type=jnp.float32)

    @pl.when(i == _NBLK - 1)
    def _():
        pooled = pooled_acc[...] / jnp.maximum(cnt_acc[...], 1.0)
        cls = cls_ref[...]
        oh_cls = (cls == jax.lax.broadcasted_iota(jnp.int32, (B, C), 1)
                  ).astype(jnp.float32)
        ce = jnp.dot(oh_cls, emb_ref[...],
                     preferred_element_type=jnp.float32)
        z = (jnp.dot(pooled, W2a_ref[...], preferred_element_type=jnp.float32)
             + jnp.dot(ce, W2b_ref[...], preferred_element_type=jnp.float32)
             + b2_ref[...])
        z = jnp.where(z > 0, z, 0.2 * z)
        res = jnp.dot(z, W3_ref[...],
                      preferred_element_type=jnp.float32) + b3_ref[...]
        out_ref[...] = res


def _dense_tail(S, dinv_col, x, batch, class_labels, W1, b1, emb, W2, b2, W3, b3):
    batch2 = batch.astype(jnp.int32).reshape(N, 1)
    cls2 = class_labels.astype(jnp.int32).reshape(B, 1)
    W2a = W2[:H]
    W2b = W2[H:]
    W3p = jnp.pad(W3, ((0, 0), (0, 127)))
    b1r = b1.reshape(1, H)
    b2r = b2.reshape(1, H)
    b3r = b3.reshape(1, 1)
    out = pl.pallas_call(
        _tail_body,
        grid=(_NBLK,),
        in_specs=[
            pl.BlockSpec((1, _ROWS, D), lambda i: (0, i, 0)),
            pl.BlockSpec((1, _ROWS, D), lambda i: (1, i, 0)),
            pl.BlockSpec((_ROWS, D), lambda i: (i, 0)),
            pl.BlockSpec((_ROWS, 1), lambda i: (i, 0)),
            pl.BlockSpec((_ROWS, 1), lambda i: (i, 0)),
            pl.BlockSpec((B, 1), lambda i: (0, 0)),
            pl.BlockSpec((D, H), lambda i: (0, 0)),
            pl.BlockSpec((1, H), lambda i: (0, 0)),
            pl.BlockSpec((C, H // 2), lambda i: (0, 0)),
            pl.BlockSpec((H, H), lambda i: (0, 0)),
            pl.BlockSpec((H // 2, H), lambda i: (0, 0)),
            pl.BlockSpec((1, H), lambda i: (0, 0)),
            pl.BlockSpec((H, 128), lambda i: (0, 0)),
            pl.BlockSpec((1, 1), lambda i: (0, 0)),
        ],
        out_specs=pl.BlockSpec((B, 128), lambda i: (0, 0)),
        out_shape=jax.ShapeDtypeStruct((B, 128), jnp.float32),
        scratch_shapes=[
            pltpu.VMEM((B, H), jnp.float32),
            pltpu.VMEM((B, 1), jnp.float32),
        ],
    )(S, S, x, dinv_col, batch2, cls2, W1, b1r, emb, W2a, W2b, b2r, W3p, b3r)
    return out[:, :1]


def kernel(x, edge_index, edge_weight, batch, class_labels,
           W1, b1, emb, W2, b2, W3, b3):
    esrc = edge_index[0].astype(jnp.int32)
    edst = edge_index[1].astype(jnp.int32)
    S, dinv = _sc_graph(x, esrc, edst, edge_weight)
    dinv_col = dinv[0, :N].reshape(N, 1)
    return _dense_tail(S, dinv_col, x, batch, class_labels,
                       W1, b1, emb, W2, b2, W3, b3)


# speedup vs baseline: 38.7977x; 1.0107x over previous
import dataclasses
import functools

import jax
import jax.numpy as jnp
from jax.experimental import pallas as pl
from jax.experimental.pallas import tpu as pltpu
from jax.experimental.pallas import tpu_sc as plsc

N = 10000
E = 320000
D = 128
H = 256
C = 10
B = 64

PADN = 10240
NSC = 2
NT = 16
SLICE = PADN // NT
E2 = E // NSC
EPT_DEG = E // NT
EPT_AGG = E2 // NT
CH = 80

_ROWS = 2000
_NBLK = N // _ROWS


NCHE = E // CH
NCH_SC = NCHE // NSC
NCH_T = NCH_SC // NT
NCH_DT = NCHE // NT


def _sc_graph(x, esrc, edst, ew):
    mesh = plsc.VectorSubcoreMesh(core_axis_name="c", subcore_axis_name="s")
    cp = pltpu.CompilerParams()
    if "needs_layout_passes" in pltpu.CompilerParams.__dataclass_fields__:
        cp = dataclasses.replace(cp, needs_layout_passes=False)

    @functools.partial(
        pl.kernel,
        compiler_params=cp,
        out_type=[jax.ShapeDtypeStruct((NSC, PADN, D), jnp.float32),
                  jax.ShapeDtypeStruct((NSC, PADN), jnp.float32)],
        mesh=mesh,
        scratch_types=[
            pltpu.VMEM_SHARED((PADN, D), jnp.float32),
            pltpu.VMEM_SHARED((PADN,), jnp.float32),
            pltpu.VMEM_SHARED((PADN,), jnp.float32),
            pltpu.VMEM((CH, D), jnp.float32),
            pltpu.VMEM((CH, D), jnp.float32),
            pltpu.VMEM((CH, D), jnp.float32),
            pltpu.VMEM((PADN,), jnp.float32),
            pltpu.VMEM((CH,), jnp.int32),
            pltpu.VMEM((CH,), jnp.int32),
            pltpu.VMEM((CH,), jnp.int32),
            pltpu.VMEM((CH,), jnp.int32),
            pltpu.VMEM((CH,), jnp.int32),
            pltpu.VMEM((CH,), jnp.int32),
            pltpu.VMEM((CH,), jnp.float32),
            pltpu.VMEM((CH,), jnp.float32),
            pltpu.VMEM((CH,), jnp.float32),
            pltpu.VMEM((1, CH), jnp.int32),
            pltpu.VMEM((1, CH), jnp.int32),
            pltpu.VMEM((1, CH), jnp.int32),
            pltpu.VMEM((CH,), jnp.float32),
            pltpu.VMEM((CH,), jnp.float32),
            pltpu.VMEM((CH,), jnp.float32),
            pltpu.VMEM((SLICE,), jnp.float32),
            pltpu.SemaphoreType.DMA,
            pltpu.SemaphoreType.DMA,
            pltpu.SemaphoreType.DMA,
            pltpu.SemaphoreType.DMA,
            pltpu.SemaphoreType.DMA,
            pltpu.SemaphoreType.DMA,
            pltpu.SemaphoreType.DMA,
            pltpu.SemaphoreType.DMA,
            pltpu.SemaphoreType.DMA,
        ],
    )
    def k(x_hbm, src_hbm, dst_hbm, ew_hbm, S_out, dinv_out,
          S_sh, deg_sh, dinv_sh, rows0, rows1, rows2, dinv_t,
          sb0, sb1, sb2, db0, db1, db2, ewb0, ewb1, ewb2,
          dg0, dg1, dg2, ef0, ef1, normb, degb,
          isem0, isem1, isem2, gsem0, gsem1, gsem2, ssem0, ssem1, ssem2):
        rowsL = (rows0, rows1, rows2)
        sbL = (sb0, sb1, sb2)
        dbL = (db0, db1, db2)
        ewbL = (ewb0, ewb1, ewb2)
        dgL = (dg0, dg1, dg2)
        efL = (ef0, ef1)
        isemL = (isem0, isem1, isem2)
        gsemL = (gsem0, gsem1, gsem2)
        ssemL = (ssem0, ssem1, ssem2)
        c = jax.lax.axis_index("c")
        s = jax.lax.axis_index("s")
        nb = s * SLICE
        z16 = jnp.zeros((16,), jnp.float32)

        @pl.loop(0, SLICE, step=16)
        def _(j):
            degb[pl.ds(j, 16)] = z16

        pltpu.sync_copy(degb, deg_sh.at[pl.ds(nb, SLICE)])

        @pl.loop(0, CH)
        def _(r):
            for j in range(8):
                rows0[r, pl.ds(j * 16, 16)] = z16

        for off in range(0, SLICE, CH):
            pltpu.sync_copy(rows0, S_sh.at[pl.ds(nb + off, CH)])
        plsc.subcore_barrier()

        drow0 = s * NCH_DT

        def d_start_i(t, r):
            e0 = (drow0 + t) * CH
            pltpu.async_copy(dst_hbm.at[pl.ds(e0, CH)], dbL[r], isemL[r])
            pltpu.async_copy(ew_hbm.at[pl.ds(e0, CH)], ewbL[r], isemL[r])

        def d_wait_i(t, r):
            e0 = (drow0 + t) * CH
            pltpu.make_async_copy(dst_hbm.at[pl.ds(e0, CH)], dbL[r],
                                  isemL[r]).wait()
            pltpu.make_async_copy(ew_hbm.at[pl.ds(e0, CH)], ewbL[r],
                                  isemL[r]).wait()

        def d_start_s(r):
            pltpu.async_copy(efL[r], deg_sh.at[dgL[r].at[0]], ssemL[r],
                             add=True)

        def d_wait_s(r):
            pltpu.make_async_copy(efL[r], deg_sh.at[dgL[r].at[0]],
                                  ssemL[r]).wait()

        def d_body(t, r):
            d_wait_i(t, r)

            @pl.when(t >= 2)
            def _():
                d_wait_s(r)

            @pl.loop(0, CH, step=16)
            def _(j):
                dgL[r][0, pl.ds(j, 16)] = dbL[r][pl.ds(j, 16)]
                efL[r][pl.ds(j, 16)] = ewbL[r][pl.ds(j, 16)]

            @pl.when(t + 2 < NCH_DT)
            def _():
                d_start_i(t + 2, r)

            d_start_s(r)

        d_start_i(0, 0)
        d_start_i(1, 1)

        @pl.loop(0, NCH_DT, step=2)
        def _(t):
            d_body(t, 0)
            d_body(t + 1, 1)

        d_wait_s(0)
        d_wait_s(1)
        plsc.subcore_barrier()

        pltpu.sync_copy(deg_sh.at[pl.ds(nb, SLICE)], degb)

        @pl.loop(0, SLICE, step=16)
        def _(j):
            dd = degb[pl.ds(j, 16)] + 1.0
            ii = plsc.bitcast(dd, jnp.int32)
            ii = 0x5F3759DF - (ii >> 1)
            y = plsc.bitcast(ii, jnp.float32)
            y = y * (1.5 - 0.5 * dd * y * y)
            y = y * (1.5 - 0.5 * dd * y * y)
            y = y * (1.5 - 0.5 * dd * y * y)
            degb[pl.ds(j, 16)] = y

        pltpu.sync_copy(degb, dinv_sh.at[pl.ds(nb, SLICE)])
        pltpu.sync_copy(degb, dinv_out.at[c, pl.ds(nb, SLICE)])
        plsc.subcore_barrier()
        pltpu.sync_copy(dinv_sh, dinv_t)

        arow0 = c * NCH_SC + s * NCH_T

        def a_start_i(t, r):
            e0 = (arow0 + t) * CH
            pltpu.async_copy(src_hbm.at[pl.ds(e0, CH)], sbL[r], isemL[r])
            pltpu.async_copy(dst_hbm.at[pl.ds(e0, CH)], dbL[r], isemL[r])
            pltpu.async_copy(ew_hbm.at[pl.ds(e0, CH)], ewbL[r], isemL[r])

        def a_wait_i(t, r):
            e0 = (arow0 + t) * CH
            pltpu.make_async_copy(src_hbm.at[pl.ds(e0, CH)], sbL[r],
                                  isemL[r]).wait()
            pltpu.make_async_copy(dst_hbm.at[pl.ds(e0, CH)], dbL[r],
                                  isemL[r]).wait()
            pltpu.make_async_copy(ew_hbm.at[pl.ds(e0, CH)], ewbL[r],
                                  isemL[r]).wait()

        def a_start_g(r):
            pltpu.async_copy(x_hbm.at[sbL[r]], rowsL[r], gsemL[r])

        def a_wait_g(r):
            pltpu.make_async_copy(x_hbm.at[sbL[r]], rowsL[r],
                                  gsemL[r]).wait()

        def a_start_s(r):
            pltpu.async_copy(rowsL[r], S_sh.at[dgL[r].at[0]], ssemL[r],
                             add=True)

        def a_wait_s(r):
            pltpu.make_async_copy(rowsL[r], S_sh.at[dgL[r].at[0]],
                                  ssemL[r]).wait()

        def a_compute(r):
            @pl.loop(0, CH, step=16)
            def _(j):
                si = sbL[r][pl.ds(j, 16)]
                di = dbL[r][pl.ds(j, 16)]
                ewv = ewbL[r][pl.ds(j, 16)]
                ns = plsc.load_gather(dinv_t, [si])
                nd = plsc.load_gather(dinv_t, [di])
                normb[pl.ds(j, 16)] = ewv * ns * nd
                dgL[r][0, pl.ds(j, 16)] = di

            @pl.loop(0, CH, step=16)
            def _(e0):
                v = normb[pl.ds(e0, 16)]
                for l in range(16):
                    sc_ = v[l]
                    e = e0 + l
                    for j in range(8):
                        rowsL[r][e, pl.ds(j * 16, 16)] = (
                            rowsL[r][e, pl.ds(j * 16, 16)] * sc_)

        def a_body(t, r):
            rn = (r + 1) % 3
            a_wait_i(t + 1, rn)

            @pl.when(t >= 2)
            def _():
                a_wait_s(rn)

            a_start_g(rn)
            a_wait_g(r)
            a_compute(r)
            a_start_s(r)

            @pl.when(t + 3 < NCH_T)
            def _():
                a_start_i(t + 3, r)

        a_start_i(0, 0)
        a_start_i(1, 1)
        a_wait_i(0, 0)
        a_start_g(0)
        a_start_i(2, 2)

        @pl.loop(0, NCH_T - 2, step=3)
        def _(t):
            a_body(t, 0)
            a_body(t + 1, 1)
            a_body(t + 2, 2)

        t0 = NCH_T - 2
        a_wait_i(t0 + 1, 1)
        a_wait_s(1)
        a_start_g(1)
        a_wait_g(0)
        a_compute(0)
        a_start_s(0)
        a_wait_g(1)
        a_compute(1)
        a_start_s(1)
        a_wait_s(2)
        a_wait_s(0)
        a_wait_s(1)
        plsc.subcore_barrier()

        pltpu.sync_copy(S_sh.at[pl.ds(nb, SLICE)], S_out.at[c, pl.ds(nb, SLICE)])

    return k(x, esrc, edst, ew)


def _tail_body(s0_ref, s1_ref, x_ref, dinv_ref, batch_ref, cls_ref,
               W1_ref, b1_ref, emb_ref, W2a_ref, W2b_ref, b2_ref,
               W3_ref, b3_ref, out_ref, pooled_acc, cnt_acc):
    i = pl.program_id(0)

    @pl.when(i == 0)
    def _():
        pooled_acc[...] = jnp.zeros_like(pooled_acc)
        cnt_acc[...] = jnp.zeros_like(cnt_acc)

    d = dinv_ref[...]
    agg = s0_ref[0] + s1_ref[0] + x_ref[...] * (d * d)
    conv = jnp.dot(agg, W1_ref[...],
                   preferred_element_type=jnp.float32) + b1_ref[...]
    out = jnp.where(conv > 0, conv, 0.2 * conv)
    seg = batch_ref[...]
    onehot = (seg == jax.lax.broadcasted_iota(jnp.int32, (_ROWS, B), 1)
              ).astype(jnp.float32)
    pooled_acc[...] += jax.lax.dot_general(
        onehot, out, (((0,), (0,)), ((), ())),
        preferred_element_type=jnp.float32)
    cnt_acc[...] += jax.lax.dot_general(
        onehot, jnp.ones((_ROWS, 1), jnp.float32), (((0,), (0,)), ((), ())),
        preferred_element_type=jnp.float32)

    @pl.when(i == _NBLK - 1)
    def _():
        pooled = pooled_acc[...] / jnp.maximum(cnt_acc[...], 1.0)
        cls = cls_ref[...]
        oh_cls = (cls == jax.lax.broadcasted_iota(jnp.int32, (B, C), 1)
                  ).astype(jnp.float32)
        ce = jnp.dot(oh_cls, emb_ref[...],
                     preferred_element_type=jnp.float32)
        z = (jnp.dot(pooled, W2a_ref[...], preferred_element_type=jnp.float32)
             + jnp.dot(ce, W2b_ref[...], preferred_element_type=jnp.float32)
             + b2_ref[...])
        z = jnp.where(z > 0, z, 0.2 * z)
        res = jnp.dot(z, W3_ref[...],
                      preferred_element_type=jnp.float32) + b3_ref[...]
        out_ref[...] = res


def _dense_tail(S, dinv_col, x, batch, class_labels, W1, b1, emb, W2, b2, W3, b3):
    batch2 = batch.astype(jnp.int32).reshape(N, 1)
    cls2 = class_labels.astype(jnp.int32).reshape(B, 1)
    W2a = W2[:H]
    W2b = W2[H:]
    W3p = jnp.pad(W3, ((0, 0), (0, 127)))
    b1r = b1.reshape(1, H)
    b2r = b2.reshape(1, H)
    b3r = b3.reshape(1, 1)
    out = pl.pallas_call(
        _tail_body,
        grid=(_NBLK,),
        in_specs=[
            pl.BlockSpec((1, _ROWS, D), lambda i: (0, i, 0)),
            pl.BlockSpec((1, _ROWS, D), lambda i: (1, i, 0)),
            pl.BlockSpec((_ROWS, D), lambda i: (i, 0)),
            pl.BlockSpec((_ROWS, 1), lambda i: (i, 0)),
            pl.BlockSpec((_ROWS, 1), lambda i: (i, 0)),
            pl.BlockSpec((B, 1), lambda i: (0, 0)),
            pl.BlockSpec((D, H), lambda i: (0, 0)),
            pl.BlockSpec((1, H), lambda i: (0, 0)),
            pl.BlockSpec((C, H // 2), lambda i: (0, 0)),
            pl.BlockSpec((H, H), lambda i: (0, 0)),
            pl.BlockSpec((H // 2, H), lambda i: (0, 0)),
            pl.BlockSpec((1, H), lambda i: (0, 0)),
            pl.BlockSpec((H, 128), lambda i: (0, 0)),
            pl.BlockSpec((1, 1), lambda i: (0, 0)),
        ],
        out_specs=pl.BlockSpec((B, 128), lambda i: (0, 0)),
        out_shape=jax.ShapeDtypeStruct((B, 128), jnp.float32),
        scratch_shapes=[
            pltpu.VMEM((B, H), jnp.float32),
            pltpu.VMEM((B, 1), jnp.float32),
        ],
    )(S, S, x, dinv_col, batch2, cls2, W1, b1r, emb, W2a, W2b, b2r, W3p, b3r)
    return out[:, :1]


def kernel(x, edge_index, edge_weight, batch, class_labels,
           W1, b1, emb, W2, b2, W3, b3):
    esrc = edge_index[0].astype(jnp.int32)
    edst = edge_index[1].astype(jnp.int32)
    S, dinv = _sc_graph(x, esrc, edst, edge_weight)
    dinv_col = dinv[0, :N].reshape(N, 1)
    return _dense_tail(S, dinv_col, x, batch, class_labels,
                       W1, b1, emb, W2, b2, W3, b3)
